# initial kernel scaffold (unmeasured)
import jax
import jax.numpy as jnp
from jax import lax
from jax.experimental import pallas as pl
from jax.experimental.pallas import tpu as pltpu

N_DEV = 4


def kernel(x, pi):
    def body(x_ref, pi_ref, out_ref, send_sem, recv_sem):
        my_pos = lax.axis_index("i")
        dst = pi_ref[my_pos]
        rdma = pltpu.make_async_remote_copy(
            src_ref=x_ref,
            dst_ref=out_ref,
            send_sem=send_sem,
            recv_sem=recv_sem,
            device_id=dst,
            device_id_type=pl.DeviceIdType.LOGICAL,
        )
        rdma.start()
        rdma.wait()

    return pl.pallas_call(
        body,
        out_shape=jax.ShapeDtypeStruct(x.shape, x.dtype),
        in_specs=[
            pl.BlockSpec(memory_space=pltpu.ANY),
            pl.BlockSpec(memory_space=pltpu.SMEM),
        ],
        out_specs=pl.BlockSpec(memory_space=pltpu.ANY),
        scratch_shapes=[
            pltpu.SemaphoreType.DMA,
            pltpu.SemaphoreType.DMA,
        ],
        compiler_params=pltpu.CompilerParams(has_side_effects=True),
    )(x, pi)


# baseline (device time: 391195 ns/iter reference)
import jax
import jax.numpy as jnp
from jax import lax
from jax.experimental import pallas as pl
from jax.experimental.pallas import tpu as pltpu

N_DEV = 4


def kernel(x, pi):
    def body(x_ref, pi_ref, out_ref, send_sem, recv_sem):
        my_pos = lax.axis_index("i")
        dst = pi_ref[my_pos]
        rdma = pltpu.make_async_remote_copy(
            src_ref=x_ref,
            dst_ref=out_ref,
            send_sem=send_sem,
            recv_sem=recv_sem,
            device_id=dst,
            device_id_type=pl.DeviceIdType.LOGICAL,
        )
        rdma.start()
        rdma.wait()

    return pl.pallas_call(
        body,
        out_shape=jax.ShapeDtypeStruct(x.shape, x.dtype),
        in_specs=[
            pl.BlockSpec(memory_space=pl.ANY),
            pl.BlockSpec(memory_space=pltpu.SMEM),
        ],
        out_specs=pl.BlockSpec(memory_space=pl.ANY),
        scratch_shapes=[
            pltpu.SemaphoreType.DMA,
            pltpu.SemaphoreType.DMA,
        ],
        compiler_params=pltpu.CompilerParams(has_side_effects=True),
    )(x, pi)


# device time: 304696 ns/iter; 1.2839x vs baseline; 1.2839x over previous
import jax
import jax.numpy as jnp
from jax import lax
from jax.experimental import pallas as pl
from jax.experimental.pallas import tpu as pltpu

N_DEV = 4

DIRECT_ROWS = 3072
REV_ROWS = 1024


def kernel(x, pi):
    def body(
        x_ref,
        pi_ref,
        out_ref,
        t1,
        t2,
        d_send,
        d_recv,
        s1_send,
        s1_recv,
        s2_send,
        s2_recv,
        s3_send,
        s3_recv,
    ):
        my = lax.axis_index("i")
        dst = pi_ref[my]
        s = lax.rem(dst - my + N_DEV, N_DEV)

        @pl.when(s == 2)
        def _():
            rdma = pltpu.make_async_remote_copy(
                src_ref=x_ref,
                dst_ref=out_ref,
                send_sem=d_send,
                recv_sem=d_recv,
                device_id=dst,
                device_id_type=pl.DeviceIdType.LOGICAL,
            )
            rdma.start()
            rdma.wait()

        @pl.when(s != 2)
        def _():
            rev = lax.rem(my + N_DEV - s, N_DEV)

            direct = pltpu.make_async_remote_copy(
                src_ref=x_ref.at[:, pl.ds(0, DIRECT_ROWS), :],
                dst_ref=out_ref.at[:, pl.ds(0, DIRECT_ROWS), :],
                send_sem=d_send,
                recv_sem=d_recv,
                device_id=dst,
                device_id_type=pl.DeviceIdType.LOGICAL,
            )
            direct.start()

            st1 = pltpu.make_async_remote_copy(
                src_ref=x_ref.at[:, pl.ds(DIRECT_ROWS, REV_ROWS), :],
                dst_ref=t1,
                send_sem=s1_send,
                recv_sem=s1_recv,
                device_id=rev,
                device_id_type=pl.DeviceIdType.LOGICAL,
            )
            st1.start()
            st1.wait_recv()

            st2 = pltpu.make_async_remote_copy(
                src_ref=t1,
                dst_ref=t2,
                send_sem=s2_send,
                recv_sem=s2_recv,
                device_id=rev,
                device_id_type=pl.DeviceIdType.LOGICAL,
            )
            st2.start()
            st2.wait_recv()

            st3 = pltpu.make_async_remote_copy(
                src_ref=t2,
                dst_ref=out_ref.at[:, pl.ds(DIRECT_ROWS, REV_ROWS), :],
                send_sem=s3_send,
                recv_sem=s3_recv,
                device_id=rev,
                device_id_type=pl.DeviceIdType.LOGICAL,
            )
            st3.start()
            st3.wait_recv()

            st1.wait_send()
            st2.wait_send()
            st3.wait_send()
            direct.wait()

    return pl.pallas_call(
        body,
        out_shape=jax.ShapeDtypeStruct(x.shape, x.dtype),
        in_specs=[
            pl.BlockSpec(memory_space=pl.ANY),
            pl.BlockSpec(memory_space=pltpu.SMEM),
        ],
        out_specs=pl.BlockSpec(memory_space=pl.ANY),
        scratch_shapes=[
            pltpu.VMEM((1, REV_ROWS, 2048), jnp.float32),
            pltpu.VMEM((1, REV_ROWS, 2048), jnp.float32),
            pltpu.SemaphoreType.DMA,
            pltpu.SemaphoreType.DMA,
            pltpu.SemaphoreType.DMA,
            pltpu.SemaphoreType.DMA,
            pltpu.SemaphoreType.DMA,
            pltpu.SemaphoreType.DMA,
            pltpu.SemaphoreType.DMA,
            pltpu.SemaphoreType.DMA,
        ],
        compiler_params=pltpu.CompilerParams(has_side_effects=True),
    )(x, pi)
